# baseline (speedup 1.0000x reference)
import jax
import jax.numpy as jnp
from jax import lax
from jax.experimental import pallas as pl
from jax.experimental.pallas import tpu as pltpu, tpu_sc as plsc

NC = 2
NS = 16
NW = NC * NS
L = 16

D = 2048
ROWS = 2 * 4096
ROWS_PER_W = ROWS // NW
R = 8
CHUNKS = ROWS_PER_W // R


def _permute_body(
    x_hbm, perm_hbm, out_hbm,
    perm_v, idx_all, in0, in1, o0, o1, s_in0, s_in1, s_out0, s_out1,
):
    wid = lax.axis_index("s") * NC + lax.axis_index("c")
    base = wid * (ROWS_PER_W * D)

    def in_slice(i):
        return x_hbm.at[pl.ds(base + i * (R * D), R * D)]

    def out_slice(i):
        return out_hbm.at[pl.ds(base + i * (R * D), R * D)]

    def compute(ib, ob):
        @plsc.parallel_loop(0, R * D, step=L, unroll=8)
        def _col(pos):
            idx = idx_all[pl.ds(pos, L)]
            ob[pl.ds(pos, L)] = plsc.load_gather(ib, [idx])

    pltpu.sync_copy(perm_hbm, perm_v)

    @plsc.parallel_loop(0, D, step=L, unroll=4)
    def _pre(off):
        idx = perm_v[pl.ds(off, L)]
        for r in range(R):
            idx_all[pl.ds(r * D + off, L)] = idx + (r * D)
    pltpu.async_copy(in_slice(0), in0, s_in0)
    pltpu.async_copy(in_slice(1), in1, s_in1)

    @pl.loop(0, CHUNKS, step=2)
    def _outer(i):
        for b, (ib, ob, si, so) in enumerate(
            ((in0, o0, s_in0, s_out0), (in1, o1, s_in1, s_out1))
        ):
            ci = i + b
            pltpu.make_async_copy(in_slice(ci), ib, si).wait()

            @pl.when(ci >= 2)
            def _drain_out():
                pltpu.make_async_copy(ob, out_slice(ci - 2), so).wait()

            compute(ib, ob)
            pltpu.async_copy(ob, out_slice(ci), so)

            @pl.when(ci + 2 < CHUNKS)
            def _prefetch_in():
                pltpu.async_copy(in_slice(ci + 2), ib, si)

    pltpu.make_async_copy(o0, out_slice(CHUNKS - 2), s_out0).wait()
    pltpu.make_async_copy(o1, out_slice(CHUNKS - 1), s_out1).wait()


@jax.jit
def _permute(x_flat, perm32):
    mesh = plsc.VectorSubcoreMesh(core_axis_name="c", subcore_axis_name="s")
    return pl.kernel(
        _permute_body,
        out_type=jax.ShapeDtypeStruct((ROWS * D,), jnp.float32),
        mesh=mesh,
        compiler_params=pltpu.CompilerParams(
            needs_layout_passes=False,
            use_tc_tiling_on_sc=False,
        ),
        scratch_types=[
            pltpu.VMEM((D,), jnp.int32),
            pltpu.VMEM((R * D,), jnp.int32),
            pltpu.VMEM((R * D,), jnp.float32),
            pltpu.VMEM((R * D,), jnp.float32),
            pltpu.VMEM((R * D,), jnp.float32),
            pltpu.VMEM((R * D,), jnp.float32),
            pltpu.SemaphoreType.DMA,
            pltpu.SemaphoreType.DMA,
            pltpu.SemaphoreType.DMA,
            pltpu.SemaphoreType.DMA,
        ],
    )(x_flat, perm32)


def kernel(x, perm):
    out = _permute(x.reshape(ROWS * D), perm.astype(jnp.int32))
    return out.reshape(x.shape)


# speedup vs baseline: 2.3902x; 2.3902x over previous
import jax
import jax.numpy as jnp
from jax import lax
from jax.experimental import pallas as pl
from jax.experimental.pallas import tpu as pltpu, tpu_sc as plsc

NC = 2
NS = 16
NW = NC * NS
L = 16

D = 2048
ROWS = 2 * 4096
R = 8
BD = R * D
NBANDS = ROWS // R
BANDS_PER_W = NBANDS // NW


def _permute_body(
    x_hbm, perm_hbm, out_hbm,
    perm_v, idx_all, in0, in1, o0, o1, s_in0, s_in1, s_out0, s_out1,
):
    wid = lax.axis_index("s") * NC + lax.axis_index("c")
    band0 = wid * BANDS_PER_W

    def in_slice(i):
        return x_hbm.at[band0 + i]

    def out_slice(i):
        return out_hbm.at[band0 + i]

    pltpu.sync_copy(perm_hbm, perm_v)

    @pl.loop(0, D // 128)
    def _pre_t(t):
        @pl.loop(0, R)
        def _pre_r(rr):
            for lc in range(128 // L):
                pj = perm_v[pl.ds(t * 128 + lc * L, L)]
                src = ((pj >> 7) << 10) + (pj & 127) + (rr << 7)
                idx_all[pl.ds(t * 1024 + rr * 128 + lc * L, L)] = src

    def compute(ib, ob):
        @plsc.parallel_loop(0, BD, step=L, unroll=8)
        def _col(p):
            idx = idx_all[pl.ds(p, L)]
            ob[pl.ds(p, L)] = plsc.load_gather(ib, [idx])

    pltpu.async_copy(in_slice(0), in0, s_in0)
    pltpu.async_copy(in_slice(1), in1, s_in1)

    @pl.loop(0, BANDS_PER_W, step=2)
    def _outer(i):
        for b, (ib, ob, si, so) in enumerate(
            ((in0, o0, s_in0, s_out0), (in1, o1, s_in1, s_out1))
        ):
            ci = i + b
            pltpu.make_async_copy(in_slice(ci), ib, si).wait()

            @pl.when(ci >= 2)
            def _drain_out():
                pltpu.make_async_copy(ob, out_slice(ci - 2), so).wait()

            compute(ib, ob)
            pltpu.async_copy(ob, out_slice(ci), so)

            @pl.when(ci + 2 < BANDS_PER_W)
            def _prefetch_in():
                pltpu.async_copy(in_slice(ci + 2), ib, si)

    pltpu.make_async_copy(o0, out_slice(BANDS_PER_W - 2), s_out0).wait()
    pltpu.make_async_copy(o1, out_slice(BANDS_PER_W - 1), s_out1).wait()


@jax.jit
def _permute(x_bands, perm32):
    mesh = plsc.VectorSubcoreMesh(core_axis_name="c", subcore_axis_name="s")
    return pl.kernel(
        _permute_body,
        out_type=jax.ShapeDtypeStruct((NBANDS, BD), jnp.float32),
        mesh=mesh,
        compiler_params=pltpu.CompilerParams(
            needs_layout_passes=False,
            use_tc_tiling_on_sc=False,
        ),
        scratch_types=[
            pltpu.VMEM((D,), jnp.int32),
            pltpu.VMEM((BD,), jnp.int32),
            pltpu.VMEM((BD,), jnp.float32),
            pltpu.VMEM((BD,), jnp.float32),
            pltpu.VMEM((BD,), jnp.float32),
            pltpu.VMEM((BD,), jnp.float32),
            pltpu.SemaphoreType.DMA,
            pltpu.SemaphoreType.DMA,
            pltpu.SemaphoreType.DMA,
            pltpu.SemaphoreType.DMA,
        ],
    )(x_bands, perm32)


def kernel(x, perm):
    b0, s, _ = x.shape
    x_bands = (
        x.reshape(b0, s // R, R, D // 128, 128)
        .transpose(0, 1, 3, 2, 4)
        .reshape(NBANDS, BD)
    )
    out = _permute(x_bands, perm.astype(jnp.int32))
    return (
        out.reshape(b0, s // R, D // 128, R, 128)
        .transpose(0, 1, 3, 2, 4)
        .reshape(x.shape)
    )


# speedup vs baseline: 2.5391x; 1.0623x over previous
import jax
import jax.numpy as jnp
from jax import lax
from jax.experimental import pallas as pl
from jax.experimental.pallas import tpu as pltpu, tpu_sc as plsc

NC = 2
NS = 16
NW = NC * NS
L = 16

D = 2048
ROWS = 2 * 4096
R = 8
BD = R * D
NBANDS = ROWS // R
BANDS_PER_W = NBANDS // NW


def _permute_body(
    x_hbm, perm_hbm, out_hbm,
    perm_v, idx_all, ins, outs, s_ins, s_outs,
):
    wid = lax.axis_index("s") * NC + lax.axis_index("c")
    band0 = wid * BANDS_PER_W

    def in_slice(i):
        return x_hbm.at[band0 + i]

    def out_slice(i):
        return out_hbm.at[band0 + i]

    pltpu.sync_copy(perm_hbm, perm_v)

    @pl.loop(0, D // 128)
    def _pre_t(t):
        @pl.loop(0, R)
        def _pre_r(rr):
            for lc in range(128 // L):
                pj = perm_v[pl.ds(t * 128 + lc * L, L)]
                src = ((pj >> 7) << 10) + (pj & 127) + (rr << 7)
                idx_all[pl.ds(t * 1024 + rr * 128 + lc * L, L)] = src

    def compute(ib, ob):
        @plsc.parallel_loop(0, BD, step=L, unroll=8)
        def _col(p):
            idx = idx_all[pl.ds(p, L)]
            ob[pl.ds(p, L)] = plsc.load_gather(ib, [idx])

    NB = 3
    MAIN = (BANDS_PER_W // NB) * NB

    for b in range(NB):
        pltpu.async_copy(in_slice(b), ins[b], s_ins[b])

    def step(ci, b, prefetch):
        pltpu.make_async_copy(in_slice(ci), ins[b], s_ins[b]).wait()

        @pl.when(ci >= NB)
        def _drain_out():
            pltpu.make_async_copy(outs[b], out_slice(ci - NB), s_outs[b]).wait()

        compute(ins[b], outs[b])
        pltpu.async_copy(outs[b], out_slice(ci), s_outs[b])
        if prefetch:
            @pl.when(ci + NB < BANDS_PER_W)
            def _prefetch_in():
                pltpu.async_copy(in_slice(ci + NB), ins[b], s_ins[b])

    @pl.loop(0, MAIN, step=NB)
    def _outer(i):
        for b in range(NB):
            step(i + b, b, True)

    for ci in range(MAIN, BANDS_PER_W):
        step(ci, ci % NB, False)

    for ci in range(BANDS_PER_W - NB, BANDS_PER_W):
        b = ci % NB
        pltpu.make_async_copy(outs[b], out_slice(ci), s_outs[b]).wait()


@jax.jit
def _permute(x_bands, perm32):
    mesh = plsc.VectorSubcoreMesh(core_axis_name="c", subcore_axis_name="s")
    return pl.kernel(
        _permute_body,
        out_type=jax.ShapeDtypeStruct((NBANDS, BD), jnp.float32),
        mesh=mesh,
        compiler_params=pltpu.CompilerParams(
            needs_layout_passes=False,
            use_tc_tiling_on_sc=False,
        ),
        scratch_types=[
            pltpu.VMEM((D,), jnp.int32),
            pltpu.VMEM((BD,), jnp.int32),
            [pltpu.VMEM((BD,), jnp.float32) for _ in range(3)],
            [pltpu.VMEM((BD,), jnp.float32) for _ in range(3)],
            [pltpu.SemaphoreType.DMA for _ in range(3)],
            [pltpu.SemaphoreType.DMA for _ in range(3)],
        ],
    )(x_bands, perm32)


def kernel(x, perm):
    b0, s, _ = x.shape
    x_bands = (
        x.reshape(b0, s // R, R, D // 128, 128)
        .transpose(0, 1, 3, 2, 4)
        .reshape(NBANDS, BD)
    )
    out = _permute(x_bands, perm.astype(jnp.int32))
    return (
        out.reshape(b0, s // R, D // 128, R, 128)
        .transpose(0, 1, 3, 2, 4)
        .reshape(x.shape)
    )
